# Initial kernel scaffold; baseline (speedup 1.0000x reference)
import jax
import jax.numpy as jnp
from jax.experimental import pallas as pl


def kernel(x, edge_index, edge_weight, W0, W1, W2):
    raise NotImplementedError("write your pallas kernel here")


# speedup vs baseline: 12.1967x; 12.1967x over previous
import functools

import jax
import jax.numpy as jnp
from jax import lax
from jax.experimental import pallas as pl
from jax.experimental.pallas import tpu as pltpu
from jax.experimental.pallas import tpu_sc as plsc

N_NODES = 100000
N_EDGES = 1600000
NC, NS, L = 2, 16, 16
NW = NC * NS
D = 16
EROW = 128
KS = 8
ROWS = 12544
ZROWS = 6248
ZTAIL = N_NODES - NS * ZROWS


def _make_spmm(col_split):
    mesh = plsc.VectorSubcoreMesh(
        core_axis_name="c", subcore_axis_name="s", num_cores=NC, num_subcores=NS)
    rpw = ROWS // NS if col_split else ROWS // NW
    chunks = rpw // KS

    @functools.partial(
        pl.kernel,
        out_type=jax.ShapeDtypeStruct((NC, N_NODES, D), jnp.float32),
        mesh=mesh,
        compiler_params=pltpu.CompilerParams(use_tc_tiling_on_sc=False),
        scratch_types=[
            pltpu.VMEM_SHARED((N_NODES, D), jnp.float32),
            pltpu.VMEM((KS, EROW), jnp.int32),
            pltpu.VMEM((KS, EROW), jnp.int32),
            pltpu.VMEM((KS, EROW), jnp.float32),
            pltpu.VMEM((KS, EROW, D), jnp.float32),
            pltpu.SemaphoreType.DMA,
        ],
    )
    def spmm(h_hbm, src_hbm, dst_hbm, w_hbm, zero_hbm, parts_hbm,
             acc, srcv, dstv, wv, msg, gsem):
        cid = lax.axis_index("c")
        sid = lax.axis_index("s")

        pltpu.sync_copy(zero_hbm.at[pl.ds(0, ZROWS)],
                        acc.at[pl.ds(sid * ZROWS, ZROWS)])

        @pl.when(sid == NS - 1)
        def _zero_tail():
            pltpu.sync_copy(zero_hbm.at[pl.ds(0, ZTAIL)],
                            acc.at[pl.ds(NS * ZROWS, ZTAIL)])

        plsc.subcore_barrier()

        base = sid * rpw if col_split else (cid * NS + sid) * rpw
        h_view = h_hbm.at[cid] if col_split else h_hbm

        @pl.loop(0, chunks)
        def chunk_body(i):
            row0 = base + i * KS
            pltpu.sync_copy(src_hbm.at[pl.ds(row0, KS)], srcv)
            pltpu.sync_copy(dst_hbm.at[pl.ds(row0, KS)], dstv)
            pltpu.sync_copy(w_hbm.at[pl.ds(row0, KS)], wv)
            descs = [pltpu.async_copy(h_view.at[srcv.at[j]], msg.at[j], gsem)
                     for j in range(KS)]
            for dsc in descs:
                dsc.wait()
            for j in range(KS):
                @pl.loop(0, EROW // L)
                def group_body(g):
                    wvec = wv[j, pl.ds(g * L, L)]
                    for k in range(L):
                        r = g * L + k
                        msg[j, r, :] = msg[j, r, :] * wvec[k]
            for j in range(KS):
                pltpu.sync_copy(msg.at[j], acc.at[dstv.at[j]], add=True)

        plsc.subcore_barrier()
        pltpu.sync_copy(acc.at[pl.ds(sid * ZROWS, ZROWS)],
                        parts_hbm.at[cid, pl.ds(sid * ZROWS, ZROWS)])

        @pl.when(sid == NS - 1)
        def _write_tail():
            pltpu.sync_copy(acc.at[pl.ds(NS * ZROWS, ZTAIL)],
                            parts_hbm.at[cid, pl.ds(NS * ZROWS, ZTAIL)])

    return spmm


_spmm_cols = _make_spmm(True)
_spmm_rows = _make_spmm(False)

_BR = 2000


def _l0_body(x_ref, w_ref, o_ref):
    o_ref[0, ...] = jnp.dot(x_ref[...], w_ref[0],
                            preferred_element_type=jnp.float32)
    o_ref[1, ...] = jnp.dot(x_ref[...], w_ref[1],
                            preferred_element_type=jnp.float32)


def _tc_layer0(x, w0cols):
    m = x.shape[0]
    return pl.pallas_call(
        _l0_body,
        grid=(m // _BR,),
        in_specs=[pl.BlockSpec((_BR, x.shape[1]), lambda i: (i, 0)),
                  pl.BlockSpec(w0cols.shape, lambda i: (0, 0, 0))],
        out_specs=pl.BlockSpec((NC, _BR, D), lambda i: (0, i, 0)),
        out_shape=jax.ShapeDtypeStruct((NC, m, D), jnp.float32),
    )(x, w0cols)


def _relu_body(p_ref, o_ref):
    o_ref[...] = jnp.maximum(p_ref[...], 0.0)


def _tc_relu(parts):
    m = parts.shape[1]
    return pl.pallas_call(
        _relu_body,
        grid=(m // _BR,),
        in_specs=[pl.BlockSpec((NC, _BR, D), lambda i: (0, i, 0))],
        out_specs=pl.BlockSpec((NC, _BR, D), lambda i: (0, i, 0)),
        out_shape=jax.ShapeDtypeStruct(parts.shape, jnp.float32),
    )(parts)


def _mid_body(p_ref, w1_ref, w2_ref, o_ref):
    agg = jnp.concatenate([p_ref[0], p_ref[1, :, :4]], axis=1)
    h2 = jnp.maximum(jnp.dot(agg, w1_ref[...],
                             preferred_element_type=jnp.float32), 0.0)
    o_ref[...] = jnp.dot(h2, w2_ref[...], preferred_element_type=jnp.float32)


def _tc_mid(parts, w1, w2p):
    m = parts.shape[1]
    return pl.pallas_call(
        _mid_body,
        grid=(m // _BR,),
        in_specs=[pl.BlockSpec((NC, _BR, D), lambda i: (0, i, 0)),
                  pl.BlockSpec(w1.shape, lambda i: (0, 0)),
                  pl.BlockSpec(w2p.shape, lambda i: (0, 0))],
        out_specs=pl.BlockSpec((_BR, D), lambda i: (i, 0)),
        out_shape=jax.ShapeDtypeStruct((m, D), jnp.float32),
    )(parts, w1, w2p)


def _final_body(p_ref, o_ref):
    o_ref[...] = p_ref[0, :, :2] + p_ref[1, :, :2]


def _tc_final(parts):
    m = parts.shape[1]
    return pl.pallas_call(
        _final_body,
        grid=(m // _BR,),
        in_specs=[pl.BlockSpec((NC, _BR, D), lambda i: (0, i, 0))],
        out_specs=pl.BlockSpec((_BR, 2), lambda i: (i, 0)),
        out_shape=jax.ShapeDtypeStruct((m, 2), jnp.float32),
    )(parts)


def kernel(x, edge_index, edge_weight, W0, W1, W2):
    src = edge_index[0].astype(jnp.int32)
    dst = edge_index[1].astype(jnp.int32)
    pad = ROWS * EROW - N_EDGES
    src2 = jnp.pad(src, (0, pad)).reshape(ROWS, EROW)
    dst2 = jnp.pad(dst, (0, pad)).reshape(ROWS, EROW)
    w2 = jnp.pad(edge_weight, (0, pad)).reshape(ROWS, EROW)
    z16 = jnp.zeros((ZROWS, D), jnp.float32)
    w0cols = jnp.stack([W0[:, :D], jnp.pad(W0[:, D:], ((0, 0), (0, 2 * D - 20)))])
    w2p = jnp.pad(W2, ((0, 0), (0, D - 2)))

    pre0 = _tc_layer0(x, w0cols)
    p1 = _spmm_cols(pre0, src2, dst2, w2, z16)
    h1 = _tc_relu(p1)
    p2 = _spmm_cols(h1, src2, dst2, w2, z16)
    pre2 = _tc_mid(p2, W1, w2p)
    p3 = _spmm_rows(pre2, src2, dst2, w2, z16)
    return _tc_final(p3)


# speedup vs baseline: 13.1009x; 1.0741x over previous
import functools

import jax
import jax.numpy as jnp
from jax import lax
from jax.experimental import pallas as pl
from jax.experimental.pallas import tpu as pltpu
from jax.experimental.pallas import tpu_sc as plsc

N_NODES = 100000
N_EDGES = 1600000
NC, NS, L = 2, 16, 16
NW = NC * NS
D = 16
EROW = 128
KS = 8
ROWS = 12544
ZROWS = 6248
ZTAIL = N_NODES - NS * ZROWS


def _make_spmm(col_split, relu_out=False):
    mesh = plsc.VectorSubcoreMesh(
        core_axis_name="c", subcore_axis_name="s", num_cores=NC, num_subcores=NS)
    rpw = ROWS // NS if col_split else ROWS // NW
    chunks = rpw // KS

    @functools.partial(
        pl.kernel,
        out_type=jax.ShapeDtypeStruct((NC, N_NODES, D), jnp.float32),
        mesh=mesh,
        compiler_params=pltpu.CompilerParams(use_tc_tiling_on_sc=False),
        scratch_types=[
            pltpu.VMEM_SHARED((N_NODES, D), jnp.float32),
            pltpu.VMEM((KS, EROW), jnp.int32),
            pltpu.VMEM((KS, EROW), jnp.int32),
            pltpu.VMEM((KS, EROW), jnp.float32),
            pltpu.VMEM((KS, EROW, D), jnp.float32),
            pltpu.SemaphoreType.DMA,
        ],
    )
    def spmm(h_hbm, src_hbm, dst_hbm, w_hbm, zero_hbm, parts_hbm,
             acc, srcv, dstv, wv, msg, gsem):
        cid = lax.axis_index("c")
        sid = lax.axis_index("s")

        pltpu.sync_copy(zero_hbm.at[pl.ds(0, ZROWS)],
                        acc.at[pl.ds(sid * ZROWS, ZROWS)])

        @pl.when(sid == NS - 1)
        def _zero_tail():
            pltpu.sync_copy(zero_hbm.at[pl.ds(0, ZTAIL)],
                            acc.at[pl.ds(NS * ZROWS, ZTAIL)])

        plsc.subcore_barrier()

        base = sid * rpw if col_split else (cid * NS + sid) * rpw
        h_view = h_hbm.at[cid] if col_split else h_hbm

        @pl.loop(0, chunks)
        def chunk_body(i):
            row0 = base + i * KS
            pltpu.sync_copy(src_hbm.at[pl.ds(row0, KS)], srcv)
            pltpu.sync_copy(dst_hbm.at[pl.ds(row0, KS)], dstv)
            pltpu.sync_copy(w_hbm.at[pl.ds(row0, KS)], wv)
            descs = [pltpu.async_copy(h_view.at[srcv.at[j]], msg.at[j], gsem)
                     for j in range(KS)]
            for dsc in descs:
                dsc.wait()
            for j in range(KS):
                @pl.loop(0, EROW // L)
                def group_body(g):
                    wvec = wv[j, pl.ds(g * L, L)]
                    for k in range(L):
                        r = g * L + k
                        msg[j, r, :] = msg[j, r, :] * wvec[k]
            for j in range(KS):
                pltpu.sync_copy(msg.at[j], acc.at[dstv.at[j]], add=True)

        plsc.subcore_barrier()

        if relu_out:
            def _relu_block(off, nrows):
                pltpu.sync_copy(acc.at[pl.ds(off, nrows)],
                                msg.at[0, pl.ds(0, nrows)])

                @pl.loop(0, nrows)
                def _relu_row(r):
                    msg[0, r, :] = jnp.maximum(msg[0, r, :], 0.0)

                pltpu.sync_copy(msg.at[0, pl.ds(0, nrows)],
                                parts_hbm.at[cid, pl.ds(off, nrows)])

            nb, rem = ZROWS // EROW, ZROWS % EROW

            @pl.loop(0, nb)
            def _relu_full(b):
                _relu_block(sid * ZROWS + b * EROW, EROW)

            _relu_block(sid * ZROWS + nb * EROW, rem)

            @pl.when(sid == NS - 1)
            def _relu_tail():
                _relu_block(NS * ZROWS, ZTAIL)
        else:
            pltpu.sync_copy(acc.at[pl.ds(sid * ZROWS, ZROWS)],
                            parts_hbm.at[cid, pl.ds(sid * ZROWS, ZROWS)])

            @pl.when(sid == NS - 1)
            def _write_tail():
                pltpu.sync_copy(acc.at[pl.ds(NS * ZROWS, ZTAIL)],
                                parts_hbm.at[cid, pl.ds(NS * ZROWS, ZTAIL)])

    return spmm


_spmm_cols = _make_spmm(True)
_spmm_cols_relu = _make_spmm(True, relu_out=True)
_spmm_rows = _make_spmm(False)

_BR = 2000


def _l0_body(x_ref, w_ref, o_ref):
    o_ref[0, ...] = jnp.dot(x_ref[...], w_ref[0],
                            preferred_element_type=jnp.float32)
    o_ref[1, ...] = jnp.dot(x_ref[...], w_ref[1],
                            preferred_element_type=jnp.float32)


def _tc_layer0(x, w0cols):
    m = x.shape[0]
    return pl.pallas_call(
        _l0_body,
        grid=(m // _BR,),
        in_specs=[pl.BlockSpec((_BR, x.shape[1]), lambda i: (i, 0)),
                  pl.BlockSpec(w0cols.shape, lambda i: (0, 0, 0))],
        out_specs=pl.BlockSpec((NC, _BR, D), lambda i: (0, i, 0)),
        out_shape=jax.ShapeDtypeStruct((NC, m, D), jnp.float32),
    )(x, w0cols)


def _relu_body(p_ref, o_ref):
    o_ref[...] = jnp.maximum(p_ref[...], 0.0)


def _tc_relu(parts):
    m = parts.shape[1]
    return pl.pallas_call(
        _relu_body,
        grid=(m // _BR,),
        in_specs=[pl.BlockSpec((NC, _BR, D), lambda i: (0, i, 0))],
        out_specs=pl.BlockSpec((NC, _BR, D), lambda i: (0, i, 0)),
        out_shape=jax.ShapeDtypeStruct(parts.shape, jnp.float32),
    )(parts)


def _mid_body(p_ref, w1_ref, w2_ref, o_ref):
    agg = jnp.concatenate([p_ref[0], p_ref[1, :, :4]], axis=1)
    h2 = jnp.maximum(jnp.dot(agg, w1_ref[...],
                             preferred_element_type=jnp.float32), 0.0)
    o_ref[...] = jnp.dot(h2, w2_ref[...], preferred_element_type=jnp.float32)


def _tc_mid(parts, w1, w2p):
    m = parts.shape[1]
    return pl.pallas_call(
        _mid_body,
        grid=(m // _BR,),
        in_specs=[pl.BlockSpec((NC, _BR, D), lambda i: (0, i, 0)),
                  pl.BlockSpec(w1.shape, lambda i: (0, 0)),
                  pl.BlockSpec(w2p.shape, lambda i: (0, 0))],
        out_specs=pl.BlockSpec((_BR, D), lambda i: (i, 0)),
        out_shape=jax.ShapeDtypeStruct((m, D), jnp.float32),
    )(parts, w1, w2p)


def _final_body(p_ref, o_ref):
    o_ref[...] = p_ref[0, :, :2] + p_ref[1, :, :2]


def _tc_final(parts):
    m = parts.shape[1]
    return pl.pallas_call(
        _final_body,
        grid=(m // _BR,),
        in_specs=[pl.BlockSpec((NC, _BR, D), lambda i: (0, i, 0))],
        out_specs=pl.BlockSpec((_BR, 2), lambda i: (i, 0)),
        out_shape=jax.ShapeDtypeStruct((m, 2), jnp.float32),
    )(parts)


def kernel(x, edge_index, edge_weight, W0, W1, W2):
    src = edge_index[0].astype(jnp.int32)
    dst = edge_index[1].astype(jnp.int32)
    pad = ROWS * EROW - N_EDGES
    src2 = jnp.pad(src, (0, pad)).reshape(ROWS, EROW)
    dst2 = jnp.pad(dst, (0, pad)).reshape(ROWS, EROW)
    w2 = jnp.pad(edge_weight, (0, pad)).reshape(ROWS, EROW)
    z16 = jnp.zeros((ZROWS, D), jnp.float32)
    w0cols = jnp.stack([W0[:, :D], jnp.pad(W0[:, D:], ((0, 0), (0, 2 * D - 20)))])
    w2p = jnp.pad(W2, ((0, 0), (0, D - 2)))

    pre0 = _tc_layer0(x, w0cols)
    h1 = _spmm_cols_relu(pre0, src2, dst2, w2, z16)
    p2 = _spmm_cols(h1, src2, dst2, w2, z16)
    pre2 = _tc_mid(p2, W1, w2p)
    p3 = _spmm_rows(pre2, src2, dst2, w2, z16)
    return _tc_final(p3)


# speedup vs baseline: 13.8563x; 1.0577x over previous
import functools

import jax
import jax.numpy as jnp
from jax import lax
from jax.experimental import pallas as pl
from jax.experimental.pallas import tpu as pltpu
from jax.experimental.pallas import tpu_sc as plsc

N_NODES = 100000
N_EDGES = 1600000
NC, NS, L = 2, 16, 16
NW = NC * NS
D = 16
EROW = 128
KS = 4
ROWS = 12544
ZROWS = 6248
ZTAIL = N_NODES - NS * ZROWS


def _make_spmm(col_split, relu_out=False):
    mesh = plsc.VectorSubcoreMesh(
        core_axis_name="c", subcore_axis_name="s", num_cores=NC, num_subcores=NS)
    rpw = ROWS // NS if col_split else ROWS // NW
    chunks = rpw // KS

    @functools.partial(
        pl.kernel,
        out_type=jax.ShapeDtypeStruct((NC, N_NODES, D), jnp.float32),
        mesh=mesh,
        compiler_params=pltpu.CompilerParams(use_tc_tiling_on_sc=False),
        scratch_types=[
            pltpu.VMEM_SHARED((N_NODES, D), jnp.float32),
            pltpu.VMEM((2, KS, EROW), jnp.int32),
            pltpu.VMEM((2, KS, EROW), jnp.int32),
            pltpu.VMEM((2, KS, EROW), jnp.float32),
            pltpu.VMEM((2, KS, EROW, D), jnp.float32),
            pltpu.SemaphoreType.DMA,
            pltpu.SemaphoreType.DMA,
        ],
    )
    def spmm(h_hbm, src_hbm, dst_hbm, w_hbm, zero_hbm, parts_hbm,
             acc, srcv, dstv, wv, msg, gsem0, gsem1):
        cid = lax.axis_index("c")
        sid = lax.axis_index("s")
        gsems = (gsem0, gsem1)

        base = sid * rpw if col_split else (cid * NS + sid) * rpw
        h_view = h_hbm.at[cid] if col_split else h_hbm

        def load_idx(b, c):
            row0 = base + c * KS
            pltpu.sync_copy(src_hbm.at[pl.ds(row0, KS)], srcv.at[b])
            pltpu.sync_copy(dst_hbm.at[pl.ds(row0, KS)], dstv.at[b])
            pltpu.sync_copy(w_hbm.at[pl.ds(row0, KS)], wv.at[b])

        def fire(b):
            for j in range(KS):
                pltpu.async_copy(h_view.at[srcv.at[b, j]], msg.at[b, j],
                                 gsems[b])

        def drain(b):
            for j in range(KS):
                pltpu.make_async_copy(h_view.at[srcv.at[b, j]],
                                      msg.at[b, j], gsems[b]).wait()

        pltpu.sync_copy(zero_hbm.at[pl.ds(0, ZROWS)],
                        acc.at[pl.ds(sid * ZROWS, ZROWS)])

        @pl.when(sid == NS - 1)
        def _zero_tail():
            pltpu.sync_copy(zero_hbm.at[pl.ds(0, ZTAIL)],
                            acc.at[pl.ds(NS * ZROWS, ZTAIL)])

        load_idx(0, 0)
        fire(0)
        load_idx(1, 1)
        fire(1)

        plsc.subcore_barrier()

        @pl.loop(0, chunks, step=2)
        def chunk_body(i):
            for b in range(2):
                c = i + b
                drain(b)
                for j in range(KS):
                    @pl.loop(0, EROW // L)
                    def group_body(g):
                        wvec = wv[b, j, pl.ds(g * L, L)]
                        for k in range(L):
                            r = g * L + k
                            msg[b, j, r, :] = msg[b, j, r, :] * wvec[k]
                for j in range(KS):
                    pltpu.sync_copy(msg.at[b, j], acc.at[dstv.at[b, j]],
                                    add=True)
                @pl.when(c + 2 < chunks)
                def _prefetch():
                    load_idx(b, c + 2)
                    fire(b)

        plsc.subcore_barrier()

        if relu_out:
            def _relu_block(off, nrows):
                pltpu.sync_copy(acc.at[pl.ds(off, nrows)],
                                msg.at[0, 0, pl.ds(0, nrows)])

                @pl.loop(0, nrows)
                def _relu_row(r):
                    msg[0, 0, r, :] = jnp.maximum(msg[0, 0, r, :], 0.0)

                pltpu.sync_copy(msg.at[0, 0, pl.ds(0, nrows)],
                                parts_hbm.at[cid, pl.ds(off, nrows)])

            nb, rem = ZROWS // EROW, ZROWS % EROW

            @pl.loop(0, nb)
            def _relu_full(b):
                _relu_block(sid * ZROWS + b * EROW, EROW)

            _relu_block(sid * ZROWS + nb * EROW, rem)

            @pl.when(sid == NS - 1)
            def _relu_tail():
                _relu_block(NS * ZROWS, ZTAIL)
        else:
            pltpu.sync_copy(acc.at[pl.ds(sid * ZROWS, ZROWS)],
                            parts_hbm.at[cid, pl.ds(sid * ZROWS, ZROWS)])

            @pl.when(sid == NS - 1)
            def _write_tail():
                pltpu.sync_copy(acc.at[pl.ds(NS * ZROWS, ZTAIL)],
                                parts_hbm.at[cid, pl.ds(NS * ZROWS, ZTAIL)])

    return spmm


_spmm_cols = _make_spmm(True)
_spmm_cols_relu = _make_spmm(True, relu_out=True)
_spmm_rows = _make_spmm(False)

_BR = 2000


def _l0_body(x_ref, w_ref, o_ref):
    o_ref[0, ...] = jnp.dot(x_ref[...], w_ref[0],
                            preferred_element_type=jnp.float32)
    o_ref[1, ...] = jnp.dot(x_ref[...], w_ref[1],
                            preferred_element_type=jnp.float32)


def _tc_layer0(x, w0cols):
    m = x.shape[0]
    return pl.pallas_call(
        _l0_body,
        grid=(m // _BR,),
        in_specs=[pl.BlockSpec((_BR, x.shape[1]), lambda i: (i, 0)),
                  pl.BlockSpec(w0cols.shape, lambda i: (0, 0, 0))],
        out_specs=pl.BlockSpec((NC, _BR, D), lambda i: (0, i, 0)),
        out_shape=jax.ShapeDtypeStruct((NC, m, D), jnp.float32),
    )(x, w0cols)


def _relu_body(p_ref, o_ref):
    o_ref[...] = jnp.maximum(p_ref[...], 0.0)


def _tc_relu(parts):
    m = parts.shape[1]
    return pl.pallas_call(
        _relu_body,
        grid=(m // _BR,),
        in_specs=[pl.BlockSpec((NC, _BR, D), lambda i: (0, i, 0))],
        out_specs=pl.BlockSpec((NC, _BR, D), lambda i: (0, i, 0)),
        out_shape=jax.ShapeDtypeStruct(parts.shape, jnp.float32),
    )(parts)


def _mid_body(p_ref, w1_ref, w2_ref, o_ref):
    agg = jnp.concatenate([p_ref[0], p_ref[1, :, :4]], axis=1)
    h2 = jnp.maximum(jnp.dot(agg, w1_ref[...],
                             preferred_element_type=jnp.float32), 0.0)
    o_ref[...] = jnp.dot(h2, w2_ref[...], preferred_element_type=jnp.float32)


def _tc_mid(parts, w1, w2p):
    m = parts.shape[1]
    return pl.pallas_call(
        _mid_body,
        grid=(m // _BR,),
        in_specs=[pl.BlockSpec((NC, _BR, D), lambda i: (0, i, 0)),
                  pl.BlockSpec(w1.shape, lambda i: (0, 0)),
                  pl.BlockSpec(w2p.shape, lambda i: (0, 0))],
        out_specs=pl.BlockSpec((_BR, D), lambda i: (i, 0)),
        out_shape=jax.ShapeDtypeStruct((m, D), jnp.float32),
    )(parts, w1, w2p)


def _final_body(p_ref, o_ref):
    o_ref[...] = p_ref[0, :, :2] + p_ref[1, :, :2]


def _tc_final(parts):
    m = parts.shape[1]
    return pl.pallas_call(
        _final_body,
        grid=(m // _BR,),
        in_specs=[pl.BlockSpec((NC, _BR, D), lambda i: (0, i, 0))],
        out_specs=pl.BlockSpec((_BR, 2), lambda i: (i, 0)),
        out_shape=jax.ShapeDtypeStruct((m, 2), jnp.float32),
    )(parts)


def kernel(x, edge_index, edge_weight, W0, W1, W2):
    src = edge_index[0].astype(jnp.int32)
    dst = edge_index[1].astype(jnp.int32)
    pad = ROWS * EROW - N_EDGES
    src2 = jnp.pad(src, (0, pad)).reshape(ROWS, EROW)
    dst2 = jnp.pad(dst, (0, pad)).reshape(ROWS, EROW)
    w2 = jnp.pad(edge_weight, (0, pad)).reshape(ROWS, EROW)
    z16 = jnp.zeros((ZROWS, D), jnp.float32)
    w0cols = jnp.stack([W0[:, :D], jnp.pad(W0[:, D:], ((0, 0), (0, 2 * D - 20)))])
    w2p = jnp.pad(W2, ((0, 0), (0, D - 2)))

    pre0 = _tc_layer0(x, w0cols)
    h1 = _spmm_cols_relu(pre0, src2, dst2, w2, z16)
    p2 = _spmm_cols(h1, src2, dst2, w2, z16)
    pre2 = _tc_mid(p2, W1, w2p)
    p3 = _spmm_rows(pre2, src2, dst2, w2, z16)
    return _tc_final(p3)


# speedup vs baseline: 13.9038x; 1.0034x over previous
import functools

import jax
import jax.numpy as jnp
from jax import lax
from jax.experimental import pallas as pl
from jax.experimental.pallas import tpu as pltpu
from jax.experimental.pallas import tpu_sc as plsc

N_NODES = 100000
N_EDGES = 1600000
NC, NS, L = 2, 16, 16
NW = NC * NS
D = 16
EROW = 128
KS = 4
ROWS = 12544
ZROWS = 6248
ZTAIL = N_NODES - NS * ZROWS


def _make_spmm(col_split, relu_out=False, fused_two_layer=False):
    mesh = plsc.VectorSubcoreMesh(
        core_axis_name="c", subcore_axis_name="s", num_cores=NC, num_subcores=NS)
    rpw = ROWS // NS if col_split else ROWS // NW
    chunks = rpw // KS

    @functools.partial(
        pl.kernel,
        out_type=jax.ShapeDtypeStruct((NC, N_NODES, D), jnp.float32),
        mesh=mesh,
        compiler_params=pltpu.CompilerParams(use_tc_tiling_on_sc=False),
        scratch_types=[
            pltpu.VMEM_SHARED((N_NODES, D), jnp.float32),
            pltpu.VMEM((2, KS, EROW), jnp.int32),
            pltpu.VMEM((2, KS, EROW), jnp.int32),
            pltpu.VMEM((2, KS, EROW), jnp.float32),
            pltpu.VMEM((2, KS, EROW, D), jnp.float32),
            pltpu.SemaphoreType.DMA,
            pltpu.SemaphoreType.DMA,
        ],
    )
    def spmm(h_hbm, src_hbm, dst_hbm, w_hbm, zero_hbm, parts_hbm,
             acc, srcv, dstv, wv, msg, gsem0, gsem1):
        cid = lax.axis_index("c")
        sid = lax.axis_index("s")
        gsems = (gsem0, gsem1)

        base = sid * rpw if col_split else (cid * NS + sid) * rpw
        h_view = h_hbm.at[cid] if col_split else h_hbm

        def load_idx(b, c):
            row0 = base + c * KS
            pltpu.sync_copy(src_hbm.at[pl.ds(row0, KS)], srcv.at[b])
            pltpu.sync_copy(dst_hbm.at[pl.ds(row0, KS)], dstv.at[b])
            pltpu.sync_copy(w_hbm.at[pl.ds(row0, KS)], wv.at[b])

        def fire(b):
            for j in range(KS):
                pltpu.async_copy(h_view.at[srcv.at[b, j]], msg.at[b, j],
                                 gsems[b])

        def drain(b):
            for j in range(KS):
                pltpu.make_async_copy(h_view.at[srcv.at[b, j]],
                                      msg.at[b, j], gsems[b]).wait()

        pltpu.sync_copy(zero_hbm.at[pl.ds(0, ZROWS)],
                        acc.at[pl.ds(sid * ZROWS, ZROWS)])

        @pl.when(sid == NS - 1)
        def _zero_tail():
            pltpu.sync_copy(zero_hbm.at[pl.ds(0, ZTAIL)],
                            acc.at[pl.ds(NS * ZROWS, ZTAIL)])

        load_idx(0, 0)
        fire(0)
        load_idx(1, 1)
        fire(1)

        plsc.subcore_barrier()

        @pl.loop(0, chunks, step=2)
        def chunk_body(i):
            for b in range(2):
                c = i + b
                drain(b)
                for j in range(KS):
                    @pl.loop(0, EROW // L)
                    def group_body(g):
                        wvec = wv[b, j, pl.ds(g * L, L)]
                        for k in range(L):
                            r = g * L + k
                            msg[b, j, r, :] = msg[b, j, r, :] * wvec[k]
                for j in range(KS):
                    pltpu.sync_copy(msg.at[b, j], acc.at[dstv.at[b, j]],
                                    add=True)
                @pl.when(c + 2 < chunks)
                def _prefetch():
                    load_idx(b, c + 2)
                    fire(b)

        plsc.subcore_barrier()

        if relu_out:
            def _relu_block(off, nrows):
                pltpu.sync_copy(acc.at[pl.ds(off, nrows)],
                                msg.at[0, 0, pl.ds(0, nrows)])

                @pl.loop(0, nrows)
                def _relu_row(r):
                    msg[0, 0, r, :] = jnp.maximum(msg[0, 0, r, :], 0.0)

                pltpu.sync_copy(msg.at[0, 0, pl.ds(0, nrows)],
                                parts_hbm.at[cid, pl.ds(off, nrows)])

            nb, rem = ZROWS // EROW, ZROWS % EROW

            @pl.loop(0, nb)
            def _relu_full(b):
                _relu_block(sid * ZROWS + b * EROW, EROW)

            _relu_block(sid * ZROWS + nb * EROW, rem)

            @pl.when(sid == NS - 1)
            def _relu_tail():
                _relu_block(NS * ZROWS, ZTAIL)
        else:
            pltpu.sync_copy(acc.at[pl.ds(sid * ZROWS, ZROWS)],
                            parts_hbm.at[cid, pl.ds(sid * ZROWS, ZROWS)])

            @pl.when(sid == NS - 1)
            def _write_tail():
                pltpu.sync_copy(acc.at[pl.ds(NS * ZROWS, ZTAIL)],
                                parts_hbm.at[cid, pl.ds(NS * ZROWS, ZTAIL)])

    return spmm


def _make_spmm12():
    mesh = plsc.VectorSubcoreMesh(
        core_axis_name="c", subcore_axis_name="s", num_cores=NC, num_subcores=NS)
    rpw = ROWS // NS
    chunks = rpw // KS

    @functools.partial(
        pl.kernel,
        out_type=[jax.ShapeDtypeStruct((NC, N_NODES, D), jnp.float32),
                  jax.ShapeDtypeStruct((NC, N_NODES, D), jnp.float32)],
        mesh=mesh,
        compiler_params=pltpu.CompilerParams(use_tc_tiling_on_sc=False),
        scratch_types=[
            pltpu.VMEM_SHARED((N_NODES, D), jnp.float32),
            pltpu.VMEM((2, KS, EROW), jnp.int32),
            pltpu.VMEM((2, KS, EROW), jnp.int32),
            pltpu.VMEM((2, KS, EROW), jnp.float32),
            pltpu.VMEM((2, KS, EROW, D), jnp.float32),
            pltpu.SemaphoreType.DMA,
            pltpu.SemaphoreType.DMA,
        ],
    )
    def spmm12(pre_hbm, src_hbm, dst_hbm, w_hbm, zero_hbm, h1_hbm, p2_hbm,
               acc, srcv, dstv, wv, msg, gsem0, gsem1):
        cid = lax.axis_index("c")
        sid = lax.axis_index("s")
        gsems = (gsem0, gsem1)
        base = sid * rpw

        def load_idx(b, c):
            row0 = base + c * KS
            pltpu.sync_copy(src_hbm.at[pl.ds(row0, KS)], srcv.at[b])
            pltpu.sync_copy(dst_hbm.at[pl.ds(row0, KS)], dstv.at[b])
            pltpu.sync_copy(w_hbm.at[pl.ds(row0, KS)], wv.at[b])

        def fire(hv, b):
            for j in range(KS):
                pltpu.async_copy(hv.at[srcv.at[b, j]], msg.at[b, j],
                                 gsems[b])

        def drain(hv, b):
            for j in range(KS):
                pltpu.make_async_copy(hv.at[srcv.at[b, j]],
                                      msg.at[b, j], gsems[b]).wait()

        def zero_acc():
            pltpu.sync_copy(zero_hbm.at[pl.ds(0, ZROWS)],
                            acc.at[pl.ds(sid * ZROWS, ZROWS)])

            @pl.when(sid == NS - 1)
            def _zero_tail():
                pltpu.sync_copy(zero_hbm.at[pl.ds(0, ZTAIL)],
                                acc.at[pl.ds(NS * ZROWS, ZTAIL)])

        def agg_pass(hv):
            @pl.loop(0, chunks, step=2)
            def chunk_body(i):
                for b in range(2):
                    c = i + b
                    drain(hv, b)
                    for j in range(KS):
                        @pl.loop(0, EROW // L)
                        def group_body(g):
                            wvec = wv[b, j, pl.ds(g * L, L)]
                            for k in range(L):
                                r = g * L + k
                                msg[b, j, r, :] = msg[b, j, r, :] * wvec[k]
                    for j in range(KS):
                        pltpu.sync_copy(msg.at[b, j], acc.at[dstv.at[b, j]],
                                        add=True)

                    @pl.when(c + 2 < chunks)
                    def _prefetch():
                        load_idx(b, c + 2)
                        fire(hv, b)

        pre_view = pre_hbm.at[cid]
        zero_acc()
        load_idx(0, 0)
        fire(pre_view, 0)
        load_idx(1, 1)
        fire(pre_view, 1)
        plsc.subcore_barrier()
        agg_pass(pre_view)
        plsc.subcore_barrier()

        def _relu_block(off, nrows):
            pltpu.sync_copy(acc.at[pl.ds(off, nrows)],
                            msg.at[0, 0, pl.ds(0, nrows)])

            @pl.loop(0, nrows)
            def _relu_row(r):
                msg[0, 0, r, :] = jnp.maximum(msg[0, 0, r, :], 0.0)

            pltpu.sync_copy(msg.at[0, 0, pl.ds(0, nrows)],
                            h1_hbm.at[cid, pl.ds(off, nrows)])

        nb, rem = ZROWS // EROW, ZROWS % EROW

        @pl.loop(0, nb)
        def _relu_full(bk):
            _relu_block(sid * ZROWS + bk * EROW, EROW)

        _relu_block(sid * ZROWS + nb * EROW, rem)

        @pl.when(sid == NS - 1)
        def _relu_tail():
            _relu_block(NS * ZROWS, ZTAIL)

        zero_acc()
        plsc.subcore_barrier()

        h1_view = h1_hbm.at[cid]
        load_idx(0, 0)
        fire(h1_view, 0)
        load_idx(1, 1)
        fire(h1_view, 1)
        agg_pass(h1_view)
        plsc.subcore_barrier()

        pltpu.sync_copy(acc.at[pl.ds(sid * ZROWS, ZROWS)],
                        p2_hbm.at[cid, pl.ds(sid * ZROWS, ZROWS)])

        @pl.when(sid == NS - 1)
        def _write_tail():
            pltpu.sync_copy(acc.at[pl.ds(NS * ZROWS, ZTAIL)],
                            p2_hbm.at[cid, pl.ds(NS * ZROWS, ZTAIL)])

    return spmm12


_spmm12 = _make_spmm12()
_spmm_rows = _make_spmm(False)

_BR = 2000


def _l0_body(x_ref, w_ref, o_ref):
    o_ref[0, ...] = jnp.dot(x_ref[...], w_ref[0],
                            preferred_element_type=jnp.float32)
    o_ref[1, ...] = jnp.dot(x_ref[...], w_ref[1],
                            preferred_element_type=jnp.float32)


def _tc_layer0(x, w0cols):
    m = x.shape[0]
    return pl.pallas_call(
        _l0_body,
        grid=(m // _BR,),
        in_specs=[pl.BlockSpec((_BR, x.shape[1]), lambda i: (i, 0)),
                  pl.BlockSpec(w0cols.shape, lambda i: (0, 0, 0))],
        out_specs=pl.BlockSpec((NC, _BR, D), lambda i: (0, i, 0)),
        out_shape=jax.ShapeDtypeStruct((NC, m, D), jnp.float32),
    )(x, w0cols)


def _relu_body(p_ref, o_ref):
    o_ref[...] = jnp.maximum(p_ref[...], 0.0)


def _tc_relu(parts):
    m = parts.shape[1]
    return pl.pallas_call(
        _relu_body,
        grid=(m // _BR,),
        in_specs=[pl.BlockSpec((NC, _BR, D), lambda i: (0, i, 0))],
        out_specs=pl.BlockSpec((NC, _BR, D), lambda i: (0, i, 0)),
        out_shape=jax.ShapeDtypeStruct(parts.shape, jnp.float32),
    )(parts)


def _mid_body(p_ref, w1_ref, w2_ref, o_ref):
    agg = jnp.concatenate([p_ref[0], p_ref[1, :, :4]], axis=1)
    h2 = jnp.maximum(jnp.dot(agg, w1_ref[...],
                             preferred_element_type=jnp.float32), 0.0)
    o_ref[...] = jnp.dot(h2, w2_ref[...], preferred_element_type=jnp.float32)


def _tc_mid(parts, w1, w2p):
    m = parts.shape[1]
    return pl.pallas_call(
        _mid_body,
        grid=(m // _BR,),
        in_specs=[pl.BlockSpec((NC, _BR, D), lambda i: (0, i, 0)),
                  pl.BlockSpec(w1.shape, lambda i: (0, 0)),
                  pl.BlockSpec(w2p.shape, lambda i: (0, 0))],
        out_specs=pl.BlockSpec((_BR, D), lambda i: (i, 0)),
        out_shape=jax.ShapeDtypeStruct((m, D), jnp.float32),
    )(parts, w1, w2p)


def _final_body(p_ref, o_ref):
    o_ref[...] = p_ref[0, :, :2] + p_ref[1, :, :2]


def _tc_final(parts):
    m = parts.shape[1]
    return pl.pallas_call(
        _final_body,
        grid=(m // _BR,),
        in_specs=[pl.BlockSpec((NC, _BR, D), lambda i: (0, i, 0))],
        out_specs=pl.BlockSpec((_BR, 2), lambda i: (i, 0)),
        out_shape=jax.ShapeDtypeStruct((m, 2), jnp.float32),
    )(parts)


def kernel(x, edge_index, edge_weight, W0, W1, W2):
    src = edge_index[0].astype(jnp.int32)
    dst = edge_index[1].astype(jnp.int32)
    pad = ROWS * EROW - N_EDGES
    src2 = jnp.pad(src, (0, pad)).reshape(ROWS, EROW)
    dst2 = jnp.pad(dst, (0, pad)).reshape(ROWS, EROW)
    w2 = jnp.pad(edge_weight, (0, pad)).reshape(ROWS, EROW)
    z16 = jnp.zeros((ZROWS, D), jnp.float32)
    w0cols = jnp.stack([W0[:, :D], jnp.pad(W0[:, D:], ((0, 0), (0, 2 * D - 20)))])
    w2p = jnp.pad(W2, ((0, 0), (0, D - 2)))

    pre0 = _tc_layer0(x, w0cols)
    _h1, p2 = _spmm12(pre0, src2, dst2, w2, z16)
    pre2 = _tc_mid(p2, W1, w2p)
    p3 = _spmm_rows(pre2, src2, dst2, w2, z16)
    return _tc_final(p3)
